# Initial kernel scaffold; baseline (speedup 1.0000x reference)
import jax
import jax.numpy as jnp
from jax.experimental import pallas as pl


def kernel(pair_idx, token_ids, token_embedding, positional_embedding, prompt_vectors_head, prompt_vectors_mid, verb_embedding, obj_embedding):
    raise NotImplementedError("write your pallas kernel here")


# speedup vs baseline: 2.8377x; 2.8377x over previous
import jax
import jax.numpy as jnp
from jax import lax
from jax.experimental import pallas as pl
from jax.experimental.pallas import tpu as pltpu, tpu_sc as plsc

B = 4096
CTX = 77
D = 512
NH = 4
NM = 3
VSLOT = NH + 1
OSLOT = NH + 2 + NM
LANES = 16
DJ = D // LANES

_info = plsc.get_sparse_core_info()
_NC = _info.num_cores
_NS = _info.num_subcores
NW = _NC * _NS
ROWS_PER_W = B // NW
C = 32
NCHUNK = ROWS_PER_W // C


def _sc_body(tokid_hbm, tokemb_hbm, pos_hbm, ph_hbm, pm_hbm,
             verb_hbm, obj_hbm, vidx_hbm, oidx_hbm, out_hbm,
             base_v, pos_v, tokid_v, vidx_v, oidx_v, vrows_v, orows_v,
             sem1, sem2):
    wid = lax.axis_index("s") * _NC + lax.axis_index("c")

    pltpu.sync_copy(tokid_hbm, tokid_v)
    pltpu.sync_copy(pos_hbm, pos_v)
    pltpu.async_copy(tokemb_hbm.at[tokid_v], base_v.at[0], sem1).wait()
    pltpu.sync_copy(ph_hbm, base_v.at[0, pl.ds(1, NH)])
    pltpu.sync_copy(pm_hbm, base_v.at[0, pl.ds(NH + 2, NM)])

    def _add_pos(i, carry):
        for j in range(DJ):
            s = pl.ds(j * LANES, LANES)
            base_v[0, i, s] = base_v[0, i, s] + pos_v[i, s]
        return carry

    lax.fori_loop(0, CTX, _add_pos, 0)

    def _chunk(c, carry):
        off = wid * ROWS_PER_W + c * C
        pltpu.sync_copy(vidx_hbm.at[pl.ds(off, C)], vidx_v)
        pltpu.sync_copy(oidx_hbm.at[pl.ds(off, C)], oidx_v)
        cp1 = pltpu.async_copy(verb_hbm.at[vidx_v], vrows_v, sem1)
        cp2 = pltpu.async_copy(obj_hbm.at[oidx_v], orows_v, sem2)
        cp1.wait()
        cp2.wait()

        def _row(i, rcarry):
            for j in range(DJ):
                s = pl.ds(j * LANES, LANES)
                base_v[0, VSLOT, s] = vrows_v[i, s] + pos_v[VSLOT, s]
                base_v[0, OSLOT, s] = orows_v[i, s] + pos_v[OSLOT, s]
            pltpu.sync_copy(base_v, out_hbm.at[pl.ds(off + i, 1)])
            return rcarry

        lax.fori_loop(0, C, _row, 0)
        return carry

    lax.fori_loop(0, NCHUNK, _chunk, 0)


def kernel(pair_idx, token_ids, token_embedding, positional_embedding,
           prompt_vectors_head, prompt_vectors_mid, verb_embedding,
           obj_embedding):
    vidx = pair_idx[:, 0].astype(jnp.int32)
    oidx = pair_idx[:, 1].astype(jnp.int32)
    tokid = token_ids.reshape(CTX).astype(jnp.int32)
    pos = positional_embedding.reshape(CTX, D)
    verb2d = verb_embedding.reshape(-1, D)
    obj2d = obj_embedding.reshape(-1, D)

    mesh = plsc.VectorSubcoreMesh(core_axis_name="c", subcore_axis_name="s")
    f = pl.kernel(
        _sc_body,
        mesh=mesh,
        compiler_params=pltpu.CompilerParams(use_tc_tiling_on_sc=False),
        out_type=jax.ShapeDtypeStruct((B, CTX, D), jnp.float32),
        scratch_types=[
            pltpu.VMEM((1, CTX, D), jnp.float32),
            pltpu.VMEM((CTX, D), jnp.float32),
            pltpu.VMEM((CTX,), jnp.int32),
            pltpu.VMEM((C,), jnp.int32),
            pltpu.VMEM((C,), jnp.int32),
            pltpu.VMEM((C, D), jnp.float32),
            pltpu.VMEM((C, D), jnp.float32),
            pltpu.SemaphoreType.DMA,
            pltpu.SemaphoreType.DMA,
        ],
    )
    return f(tokid, token_embedding, pos, prompt_vectors_head,
             prompt_vectors_mid, verb2d, obj2d, vidx, oidx)
